# baseline (speedup 1.0000x reference)
import functools

import jax
import jax.numpy as jnp
from jax import lax
from jax.experimental import pallas as pl
from jax.experimental.pallas import tpu as pltpu
from jax.experimental.pallas import tpu_sc as plsc

D_MODEL_K = 2048
N_EXP = 16
N_TOKENS = 16384

BT = 1024

NC = 1
NS = 16
NW = NC * NS
CHUNK = N_TOKENS // NW
SUB = 128
NSUB = CHUNK // SUB
TB = 16


def _gate_matmul_body(w_ref, x_ref, out_ref):
    out_ref[...] = lax.dot_general(
        w_ref[...], x_ref[...],
        (((1,), (1,)), ((), ())),
        preferred_element_type=jnp.float32,
    )


def _gate_matmul(x, gate_W):
    n_tok = x.shape[0]
    return pl.pallas_call(
        _gate_matmul_body,
        grid=(n_tok // BT,),
        in_specs=[
            pl.BlockSpec((N_EXP, D_MODEL_K), lambda i: (0, 0)),
            pl.BlockSpec((BT, D_MODEL_K), lambda i: (i, 0)),
        ],
        out_specs=pl.BlockSpec((N_EXP, BT), lambda i: (0, i)),
        out_shape=jax.ShapeDtypeStruct((N_EXP, n_tok), jnp.float32),
    )(gate_W, x)


def _route_body(lt_hbm, b_hbm, out_hbm, idx_hbm, lt_v, b_v, out_v, idx_v):
    cid = lax.axis_index("c")
    sid = lax.axis_index("s")
    wid = sid * NC + cid
    base = wid * CHUNK

    pltpu.sync_copy(lt_hbm.at[:, pl.ds(base, CHUNK)], lt_v)
    pltpu.sync_copy(b_hbm, b_v)

    zero_f = jnp.zeros((16,), jnp.float32)
    one_f = jnp.ones((16,), jnp.float32)
    neg_inf = jnp.full((16,), -jnp.inf, jnp.float32)
    zero_i = jnp.zeros((16,), jnp.int32)

    lanes = lax.iota(jnp.int32, 16)
    bv = b_v[pl.ds(0, 16)]
    bias = [
        jnp.full((16,), jnp.sum(jnp.where(lanes == e, bv, zero_f)))
        for e in range(N_EXP)
    ]

    def block(blk, _):
        t0 = blk * TB
        m1 = lt_v[0, pl.ds(t0, TB)] + bias[0]
        i1 = zero_i
        m2 = neg_inf
        i2 = zero_i
        for e in range(1, N_EXP):
            v = lt_v[e, pl.ds(t0, TB)] + bias[e]
            gt1 = v > m1
            gt2 = v > m2
            new_m2 = jnp.where(gt1, m1, jnp.where(gt2, v, m2))
            new_i2 = jnp.where(gt1, i1, jnp.where(gt2, e, i2))
            m1 = jnp.where(gt1, v, m1)
            i1 = jnp.where(gt1, e, i1)
            m2 = new_m2
            i2 = new_i2

        p1 = one_f / (one_f + jnp.exp(m2 - m1))
        p2 = one_f - p1

        for e in range(N_EXP):
            col = jnp.where(i1 == e, p1, jnp.where(i2 == e, p2, zero_f))
            out_v[e, pl.ds(t0, TB)] = col
        idx_v[0, pl.ds(t0, TB)] = i1
        idx_v[1, pl.ds(t0, TB)] = i2
        return 0

    lax.fori_loop(0, CHUNK // TB, block, 0)

    pltpu.sync_copy(out_v, out_hbm.at[:, pl.ds(base, CHUNK)])
    pltpu.sync_copy(idx_v, idx_hbm.at[:, pl.ds(base, CHUNK)])


def _route(logits_t, gate_b):
    n_tok = logits_t.shape[1]
    mesh = plsc.VectorSubcoreMesh(core_axis_name="c", subcore_axis_name="s", num_cores=1)
    fn = functools.partial(
        pl.kernel,
        mesh=mesh,
        compiler_params=pltpu.CompilerParams(needs_layout_passes=False),
        out_type=[
            jax.ShapeDtypeStruct((N_EXP, n_tok), jnp.float32),
            jax.ShapeDtypeStruct((2, n_tok), jnp.int32),
        ],
        scratch_types=[
            pltpu.VMEM((N_EXP, CHUNK), jnp.float32),
            pltpu.VMEM((16,), jnp.float32),
            pltpu.VMEM((N_EXP, CHUNK), jnp.float32),
            pltpu.VMEM((2, CHUNK), jnp.int32),
        ],
    )(_route_body)
    return fn(logits_t, gate_b)


def kernel(x, gate_W, gate_b):
    n_tok = x.shape[0]
    logits_t = _gate_matmul(x, gate_W)
    out_t, idx_t = _route(logits_t, gate_b)
    return out_t.T, idx_t.T


# speedup vs baseline: 1.0321x; 1.0321x over previous
import functools

import jax
import jax.numpy as jnp
from jax import lax
from jax.experimental import pallas as pl
from jax.experimental.pallas import tpu as pltpu
from jax.experimental.pallas import tpu_sc as plsc

D_MODEL_K = 2048
N_EXP = 16
N_TOKENS = 16384

BT = 1024

NC = 2
NS = 16
NW = NC * NS
CHUNK = N_TOKENS // NW
SUB = 128
NSUB = CHUNK // SUB
TB = 16


def _gate_matmul_body(w_ref, x_ref, out_ref):
    out_ref[...] = lax.dot_general(
        w_ref[...], x_ref[...],
        (((1,), (1,)), ((), ())),
        preferred_element_type=jnp.float32,
    )


def _gate_matmul(x, gate_W):
    n_tok = x.shape[0]
    return pl.pallas_call(
        _gate_matmul_body,
        grid=(n_tok // BT,),
        in_specs=[
            pl.BlockSpec((N_EXP, D_MODEL_K), lambda i: (0, 0)),
            pl.BlockSpec((BT, D_MODEL_K), lambda i: (i, 0)),
        ],
        out_specs=pl.BlockSpec((N_EXP, BT), lambda i: (0, i)),
        out_shape=jax.ShapeDtypeStruct((N_EXP, n_tok), jnp.float32),
    )(gate_W, x)


def _route_body(lt_hbm, b_hbm, out_hbm, idx_hbm, lt_v, b_v, out_v, idx_v):
    cid = lax.axis_index("c")
    sid = lax.axis_index("s")
    wid = sid * NC + cid
    base = wid * CHUNK

    pltpu.sync_copy(lt_hbm.at[:, pl.ds(base, CHUNK)], lt_v)
    pltpu.sync_copy(b_hbm, b_v)

    zero_f = jnp.zeros((16,), jnp.float32)
    one_f = jnp.ones((16,), jnp.float32)
    neg_inf = jnp.full((16,), -jnp.inf, jnp.float32)
    zero_i = jnp.zeros((16,), jnp.int32)

    lanes = lax.iota(jnp.int32, 16)
    bv = b_v[pl.ds(0, 16)]
    bias = [
        jnp.full((16,), jnp.sum(jnp.where(lanes == e, bv, zero_f)))
        for e in range(N_EXP)
    ]

    def block(blk, _):
        t0 = blk * TB

        def leaf(e):
            a = lt_v[e, pl.ds(t0, TB)] + bias[e]
            b = lt_v[e + 1, pl.ds(t0, TB)] + bias[e + 1]
            gt = b > a
            m1 = jnp.where(gt, b, a)
            i1 = jnp.where(gt, e + 1, e)
            m2 = jnp.where(gt, a, b)
            i2 = jnp.where(gt, e, e + 1)
            return m1, i1, m2, i2

        def merge(p, q):
            pm1, pi1, pm2, pi2 = p
            qm1, qi1, qm2, qi2 = q
            gt = qm1 > pm1
            c2 = qm2 > pm1
            c3 = qm1 > pm2
            m1 = jnp.where(gt, qm1, pm1)
            i1 = jnp.where(gt, qi1, pi1)
            m2 = jnp.where(gt, jnp.where(c2, qm2, pm1), jnp.where(c3, qm1, pm2))
            i2 = jnp.where(gt, jnp.where(c2, qi2, pi1), jnp.where(c3, qi1, pi2))
            return m1, i1, m2, i2

        groups = [leaf(e) for e in range(0, N_EXP, 2)]
        while len(groups) > 1:
            groups = [merge(groups[k], groups[k + 1])
                      for k in range(0, len(groups), 2)]
        m1, i1, m2, i2 = groups[0]

        p1 = one_f / (one_f + jnp.exp(m2 - m1))
        p2 = one_f - p1

        for e in range(N_EXP):
            col = jnp.where(i1 == e, p1, jnp.where(i2 == e, p2, zero_f))
            out_v[e, pl.ds(t0, TB)] = col
        idx_v[0, pl.ds(t0, TB)] = i1
        idx_v[1, pl.ds(t0, TB)] = i2
        return 0

    lax.fori_loop(0, CHUNK // TB, block, 0)

    pltpu.sync_copy(out_v, out_hbm.at[:, pl.ds(base, CHUNK)])
    pltpu.sync_copy(idx_v, idx_hbm.at[:, pl.ds(base, CHUNK)])


def _route(logits_t, gate_b):
    n_tok = logits_t.shape[1]
    mesh = plsc.VectorSubcoreMesh(core_axis_name="c", subcore_axis_name="s")
    fn = functools.partial(
        pl.kernel,
        mesh=mesh,
        compiler_params=pltpu.CompilerParams(needs_layout_passes=False),
        out_type=[
            jax.ShapeDtypeStruct((N_EXP, n_tok), jnp.float32),
            jax.ShapeDtypeStruct((2, n_tok), jnp.int32),
        ],
        scratch_types=[
            pltpu.VMEM((N_EXP, CHUNK), jnp.float32),
            pltpu.VMEM((16,), jnp.float32),
            pltpu.VMEM((N_EXP, CHUNK), jnp.float32),
            pltpu.VMEM((2, CHUNK), jnp.int32),
        ],
    )(_route_body)
    return fn(logits_t, gate_b)


def kernel(x, gate_W, gate_b):
    n_tok = x.shape[0]
    logits_t = _gate_matmul(x, gate_W)
    out_t, idx_t = _route(logits_t, gate_b)
    return out_t.T, idx_t.T


# speedup vs baseline: 1.0332x; 1.0011x over previous
import functools

import jax
import jax.numpy as jnp
from jax import lax
from jax.experimental import pallas as pl
from jax.experimental.pallas import tpu as pltpu
from jax.experimental.pallas import tpu_sc as plsc

D_MODEL_K = 2048
N_EXP = 16
N_TOKENS = 16384

BT = 1024

NC = 2
NS = 16
NW = NC * NS
CHUNK = N_TOKENS // NW
SUB = 128
NSUB = CHUNK // SUB
TB = 16


def _gate_matmul_body(w_ref, x_ref, out_ref):
    lt = lax.dot_general(
        w_ref[...], x_ref[...],
        (((1,), (1,)), ((), ())),
        preferred_element_type=jnp.float32,
    )
    for w in range(BT // CHUNK):
        out_ref[w] = lt[:, w * CHUNK:(w + 1) * CHUNK]


def _gate_matmul(x, gate_W):
    n_tok = x.shape[0]
    return pl.pallas_call(
        _gate_matmul_body,
        grid=(n_tok // BT,),
        in_specs=[
            pl.BlockSpec((N_EXP, D_MODEL_K), lambda i: (0, 0)),
            pl.BlockSpec((BT, D_MODEL_K), lambda i: (i, 0)),
        ],
        out_specs=pl.BlockSpec((BT // CHUNK, N_EXP, CHUNK), lambda i: (i, 0, 0)),
        out_shape=jax.ShapeDtypeStruct((n_tok // CHUNK, N_EXP, CHUNK), jnp.float32),
    )(gate_W, x)


def _route_body(lt_hbm, b_hbm, out_hbm, idx_hbm, lt_v, b_v, out_v, idx_v):
    cid = lax.axis_index("c")
    sid = lax.axis_index("s")
    wid = sid * NC + cid
    base = wid * CHUNK

    pltpu.sync_copy(lt_hbm.at[wid], lt_v)
    pltpu.sync_copy(b_hbm, b_v)

    zero_f = jnp.zeros((16,), jnp.float32)
    one_f = jnp.ones((16,), jnp.float32)
    neg_inf = jnp.full((16,), -jnp.inf, jnp.float32)
    zero_i = jnp.zeros((16,), jnp.int32)

    lanes = lax.iota(jnp.int32, 16)
    bv = b_v[pl.ds(0, 16)]
    bias = [
        jnp.full((16,), jnp.sum(jnp.where(lanes == e, bv, zero_f)))
        for e in range(N_EXP)
    ]

    def block(blk, _):
        t0 = blk * TB

        def leaf(e):
            a = lt_v[e, pl.ds(t0, TB)] + bias[e]
            b = lt_v[e + 1, pl.ds(t0, TB)] + bias[e + 1]
            gt = b > a
            m1 = jnp.where(gt, b, a)
            i1 = jnp.where(gt, e + 1, e)
            m2 = jnp.where(gt, a, b)
            i2 = jnp.where(gt, e, e + 1)
            return m1, i1, m2, i2

        def merge(p, q):
            pm1, pi1, pm2, pi2 = p
            qm1, qi1, qm2, qi2 = q
            gt = qm1 > pm1
            c2 = qm2 > pm1
            c3 = qm1 > pm2
            m1 = jnp.where(gt, qm1, pm1)
            i1 = jnp.where(gt, qi1, pi1)
            m2 = jnp.where(gt, jnp.where(c2, qm2, pm1), jnp.where(c3, qm1, pm2))
            i2 = jnp.where(gt, jnp.where(c2, qi2, pi1), jnp.where(c3, qi1, pi2))
            return m1, i1, m2, i2

        groups = [leaf(e) for e in range(0, N_EXP, 2)]
        while len(groups) > 1:
            groups = [merge(groups[k], groups[k + 1])
                      for k in range(0, len(groups), 2)]
        m1, i1, m2, i2 = groups[0]

        p1 = one_f / (one_f + jnp.exp(m2 - m1))
        p2 = one_f - p1

        for e in range(N_EXP):
            col = jnp.where(i1 == e, p1, jnp.where(i2 == e, p2, zero_f))
            out_v[e, pl.ds(t0, TB)] = col
        idx_v[0, pl.ds(t0, TB)] = i1
        idx_v[1, pl.ds(t0, TB)] = i2
        return 0

    lax.fori_loop(0, CHUNK // TB, block, 0)

    pltpu.sync_copy(out_v, out_hbm.at[:, pl.ds(base, CHUNK)])
    pltpu.sync_copy(idx_v, idx_hbm.at[:, pl.ds(base, CHUNK)])


def _route(logits_t, gate_b):
    n_tok = logits_t.shape[0] * logits_t.shape[2]
    mesh = plsc.VectorSubcoreMesh(core_axis_name="c", subcore_axis_name="s")
    fn = functools.partial(
        pl.kernel,
        mesh=mesh,
        compiler_params=pltpu.CompilerParams(needs_layout_passes=False),
        out_type=[
            jax.ShapeDtypeStruct((N_EXP, n_tok), jnp.float32),
            jax.ShapeDtypeStruct((2, n_tok), jnp.int32),
        ],
        scratch_types=[
            pltpu.VMEM((N_EXP, CHUNK), jnp.float32),
            pltpu.VMEM((16,), jnp.float32),
            pltpu.VMEM((N_EXP, CHUNK), jnp.float32),
            pltpu.VMEM((2, CHUNK), jnp.int32),
        ],
    )(_route_body)
    return fn(logits_t, gate_b)


def kernel(x, gate_W, gate_b):
    n_tok = x.shape[0]
    logits_t = _gate_matmul(x, gate_W)
    out_t, idx_t = _route(logits_t, gate_b)
    return out_t.T, idx_t.T
